# baseline (speedup 1.0000x reference)
import jax
import jax.numpy as jnp
import numpy as np
from jax.experimental import pallas as pl
from jax.experimental.pallas import tpu as pltpu

_N_CODES = 8192
_DIM = 32
_BETA = 0.25
_ROWS = 512
_CHUNK = 2048

_DN = (((1,), (0,)), ((), ()))


def _vq_body(x_ref, ebt_ref, ecat_ref, z2_ref, e2_ref,
             xq_ref, ind_ref, loss_ref, unused_ref, cnt_ref):
    step = pl.program_id(0)
    nsteps = pl.num_programs(0)

    @pl.when(step == 0)
    def _init():
        cnt_ref[...] = jnp.zeros_like(cnt_ref)
        loss_ref[...] = jnp.zeros_like(loss_ref)
        unused_ref[...] = jnp.zeros_like(unused_ref)

    z = x_ref[...]
    zb = (2.0 * z).astype(jnp.bfloat16)
    m = jax.lax.dot_general(zb, ebt_ref[...], _DN,
                            preferred_element_type=jnp.float32)
    dist = (z2_ref[...] - m) + e2_ref[...]

    acc = jnp.full((_ROWS, 1), jnp.inf, dtype=jnp.float32)
    ind = jnp.zeros((_ROWS, 1), dtype=jnp.int32)
    for k in range(_N_CODES // _CHUNK):
        dk = jax.lax.slice(dist, (0, k * _CHUNK), (_ROWS, (k + 1) * _CHUNK))
        mnk = jnp.min(dk, axis=1, keepdims=True)
        iota = jax.lax.broadcasted_iota(jnp.int32, dk.shape, 1) + k * _CHUNK
        idxk = jnp.min(jnp.where(dk == mnk, iota, _N_CODES), axis=1,
                       keepdims=True)
        take = mnk < acc
        acc = jnp.where(take, mnk.astype(jnp.bfloat16).astype(jnp.float32),
                        acc)
        ind = jnp.where(take, idxk, ind)
    ind_ref[...] = ind

    full_iota = jax.lax.broadcasted_iota(jnp.int32, dist.shape, 1)
    oh = (full_iota == ind).astype(jnp.bfloat16)
    x3 = jax.lax.dot_general(oh, ecat_ref[...], _DN,
                             preferred_element_type=jnp.float32)
    xq = (jax.lax.slice(x3, (0, 0), (_ROWS, _DIM))
          + jax.lax.slice(x3, (0, _DIM), (_ROWS, 2 * _DIM))) \
         + jax.lax.slice(x3, (0, 2 * _DIM), (_ROWS, 3 * _DIM))
    t = xq - z
    xq_ref[...] = z + t
    loss_ref[...] = loss_ref[...] + jnp.sum(t * t)
    ones = jnp.ones((1, _ROWS), dtype=jnp.bfloat16)
    cnt_ref[...] += jax.lax.dot_general(ones, oh, _DN,
                                        preferred_element_type=jnp.float32)

    @pl.when(step == nsteps - 1)
    def _fin():
        msq = loss_ref[...] / np.float32(nsteps * _ROWS * _DIM)
        loss_ref[...] = msq + np.float32(_BETA) * msq
        unused_ref[...] = jnp.sum((cnt_ref[...] == 0.0).astype(jnp.int32),
                                  axis=(0, 1), keepdims=True)


def kernel(x, embed):
    latent = x.reshape(-1, _DIM)
    n = latent.shape[0]
    z2 = jnp.sum(latent ** 2, axis=1, keepdims=True)
    e2 = jnp.sum(embed ** 2, axis=1)[None, :]
    ebt = embed.astype(jnp.bfloat16).T
    e_hi = embed.astype(jnp.bfloat16)
    r1 = embed - e_hi.astype(jnp.float32)
    e_mid = r1.astype(jnp.bfloat16)
    e_lo = (r1 - e_mid.astype(jnp.float32)).astype(jnp.bfloat16)
    e_cat = jnp.concatenate([e_hi, e_mid, e_lo], axis=1)
    xq, ind, loss, unused = pl.pallas_call(
        _vq_body,
        grid=(n // _ROWS,),
        in_specs=[
            pl.BlockSpec((_ROWS, _DIM), lambda i: (i, 0)),
            pl.BlockSpec((_DIM, _N_CODES), lambda i: (0, 0)),
            pl.BlockSpec((_N_CODES, 3 * _DIM), lambda i: (0, 0)),
            pl.BlockSpec((_ROWS, 1), lambda i: (i, 0)),
            pl.BlockSpec((1, _N_CODES), lambda i: (0, 0)),
        ],
        out_specs=[
            pl.BlockSpec((_ROWS, _DIM), lambda i: (i, 0)),
            pl.BlockSpec((_ROWS, 1), lambda i: (i, 0)),
            pl.BlockSpec((1, 1), lambda i: (0, 0)),
            pl.BlockSpec((1, 1), lambda i: (0, 0)),
        ],
        out_shape=[
            jax.ShapeDtypeStruct((n, _DIM), jnp.float32),
            jax.ShapeDtypeStruct((n, 1), jnp.int32),
            jax.ShapeDtypeStruct((1, 1), jnp.float32),
            jax.ShapeDtypeStruct((1, 1), jnp.int32),
        ],
        scratch_shapes=[pltpu.VMEM((1, _N_CODES), jnp.float32)],
    )(latent, ebt, e_cat, z2, e2)
    x_q_st = xq.reshape(x.shape)
    quant_loss = loss[0, 0]
    unused_codes = unused[0, 0]
    embed_ind = ind.reshape(x.shape[:-1])
    return (x_q_st, quant_loss, unused_codes, embed_ind)


# speedup vs baseline: 1.0675x; 1.0675x over previous
import jax
import jax.numpy as jnp
import numpy as np
from jax.experimental import pallas as pl
from jax.experimental.pallas import tpu as pltpu

_N_CODES = 8192
_DIM = 32
_BETA = 0.25
_ROWS = 128
_CHUNK = 2048

_DN = (((1,), (0,)), ((), ()))


def _vq_body(x_ref, ebt_ref, ecat_ref, z2_ref, e2_ref,
             xq_ref, ind_ref, loss_ref, unused_ref, cnt_ref):
    step = pl.program_id(0)
    nsteps = pl.num_programs(0)

    @pl.when(step == 0)
    def _init():
        cnt_ref[...] = jnp.zeros_like(cnt_ref)
        loss_ref[...] = jnp.zeros_like(loss_ref)
        unused_ref[...] = jnp.zeros_like(unused_ref)

    z = x_ref[...]
    zb = (2.0 * z).astype(jnp.bfloat16)
    m = jax.lax.dot_general(zb, ebt_ref[...], _DN,
                            preferred_element_type=jnp.float32)
    dist = (z2_ref[...] - m) + e2_ref[...]

    acc = jnp.full((_ROWS, 1), jnp.inf, dtype=jnp.float32)
    ind = jnp.zeros((_ROWS, 1), dtype=jnp.int32)
    for k in range(_N_CODES // _CHUNK):
        dk = jax.lax.slice(dist, (0, k * _CHUNK), (_ROWS, (k + 1) * _CHUNK))
        mnk = jnp.min(dk, axis=1, keepdims=True)
        iota = jax.lax.broadcasted_iota(jnp.int32, dk.shape, 1) + k * _CHUNK
        idxk = jnp.min(jnp.where(dk == mnk, iota, _N_CODES), axis=1,
                       keepdims=True)
        take = mnk < acc
        acc = jnp.where(take, mnk.astype(jnp.bfloat16).astype(jnp.float32),
                        acc)
        ind = jnp.where(take, idxk, ind)
    ind_ref[...] = ind

    full_iota = jax.lax.broadcasted_iota(jnp.int32, dist.shape, 1)
    oh = (full_iota == ind).astype(jnp.bfloat16)
    x3 = jax.lax.dot_general(oh, ecat_ref[...], _DN,
                             preferred_element_type=jnp.float32)
    xq = (jax.lax.slice(x3, (0, 0), (_ROWS, _DIM))
          + jax.lax.slice(x3, (0, _DIM), (_ROWS, 2 * _DIM))) \
         + jax.lax.slice(x3, (0, 2 * _DIM), (_ROWS, 3 * _DIM))
    t = xq - z
    xq_ref[...] = z + t
    loss_ref[...] = loss_ref[...] + jnp.sum(t * t)
    ones = jnp.ones((1, _ROWS), dtype=jnp.bfloat16)
    cnt_ref[...] += jax.lax.dot_general(ones, oh, _DN,
                                        preferred_element_type=jnp.float32)

    @pl.when(step == nsteps - 1)
    def _fin():
        msq = loss_ref[...] / np.float32(nsteps * _ROWS * _DIM)
        loss_ref[...] = msq + np.float32(_BETA) * msq
        unused_ref[...] = jnp.sum((cnt_ref[...] == 0.0).astype(jnp.int32),
                                  axis=(0, 1), keepdims=True)


def kernel(x, embed):
    latent = x.reshape(-1, _DIM)
    n = latent.shape[0]
    z2 = jnp.sum(latent ** 2, axis=1, keepdims=True)
    e2 = jnp.sum(embed ** 2, axis=1)[None, :]
    ebt = embed.astype(jnp.bfloat16).T
    e_hi = embed.astype(jnp.bfloat16)
    r1 = embed - e_hi.astype(jnp.float32)
    e_mid = r1.astype(jnp.bfloat16)
    e_lo = (r1 - e_mid.astype(jnp.float32)).astype(jnp.bfloat16)
    e_cat = jnp.concatenate([e_hi, e_mid, e_lo], axis=1)
    xq, ind, loss, unused = pl.pallas_call(
        _vq_body,
        grid=(n // _ROWS,),
        in_specs=[
            pl.BlockSpec((_ROWS, _DIM), lambda i: (i, 0)),
            pl.BlockSpec((_DIM, _N_CODES), lambda i: (0, 0)),
            pl.BlockSpec((_N_CODES, 3 * _DIM), lambda i: (0, 0)),
            pl.BlockSpec((_ROWS, 1), lambda i: (i, 0)),
            pl.BlockSpec((1, _N_CODES), lambda i: (0, 0)),
        ],
        out_specs=[
            pl.BlockSpec((_ROWS, _DIM), lambda i: (i, 0)),
            pl.BlockSpec((_ROWS, 1), lambda i: (i, 0)),
            pl.BlockSpec((1, 1), lambda i: (0, 0)),
            pl.BlockSpec((1, 1), lambda i: (0, 0)),
        ],
        out_shape=[
            jax.ShapeDtypeStruct((n, _DIM), jnp.float32),
            jax.ShapeDtypeStruct((n, 1), jnp.int32),
            jax.ShapeDtypeStruct((1, 1), jnp.float32),
            jax.ShapeDtypeStruct((1, 1), jnp.int32),
        ],
        scratch_shapes=[pltpu.VMEM((1, _N_CODES), jnp.float32)],
    )(latent, ebt, e_cat, z2, e2)
    x_q_st = xq.reshape(x.shape)
    quant_loss = loss[0, 0]
    unused_codes = unused[0, 0]
    embed_ind = ind.reshape(x.shape[:-1])
    return (x_q_st, quant_loss, unused_codes, embed_ind)


# speedup vs baseline: 1.1471x; 1.0745x over previous
import jax
import jax.numpy as jnp
import numpy as np
from jax.experimental import pallas as pl
from jax.experimental.pallas import tpu as pltpu

_N_CODES = 8192
_DIM = 32
_BETA = 0.25
_ROWS = 256
_CHUNK = 2048

_DN = (((1,), (0,)), ((), ()))


def _vq_body(x_ref, ebt_ref, ecat_ref, z2_ref, e2_ref,
             xq_ref, ind_ref, loss_ref, unused_ref, cnt_ref):
    step = pl.program_id(0)
    nsteps = pl.num_programs(0)

    @pl.when(step == 0)
    def _init():
        cnt_ref[...] = jnp.zeros_like(cnt_ref)
        loss_ref[...] = jnp.zeros_like(loss_ref)
        unused_ref[...] = jnp.zeros_like(unused_ref)

    z = x_ref[...]
    zb = (2.0 * z).astype(jnp.bfloat16)
    m = jax.lax.dot_general(zb, ebt_ref[...], _DN,
                            preferred_element_type=jnp.float32)
    dist = (z2_ref[...] - m) + e2_ref[...]

    acc = jnp.full((_ROWS, 1), jnp.inf, dtype=jnp.float32)
    ind = jnp.zeros((_ROWS, 1), dtype=jnp.int32)
    for k in range(_N_CODES // _CHUNK):
        dk = jax.lax.slice(dist, (0, k * _CHUNK), (_ROWS, (k + 1) * _CHUNK))
        mnk = jnp.min(dk, axis=1, keepdims=True)
        iota = jax.lax.broadcasted_iota(jnp.int32, dk.shape, 1) + k * _CHUNK
        idxk = jnp.min(jnp.where(dk == mnk, iota, _N_CODES), axis=1,
                       keepdims=True)
        take = mnk < acc
        acc = jnp.where(take, mnk.astype(jnp.bfloat16).astype(jnp.float32),
                        acc)
        ind = jnp.where(take, idxk, ind)
    ind_ref[...] = ind

    full_iota = jax.lax.broadcasted_iota(jnp.int32, dist.shape, 1)
    oh = (full_iota == ind).astype(jnp.bfloat16)
    x3 = jax.lax.dot_general(oh, ecat_ref[...], _DN,
                             preferred_element_type=jnp.float32)
    xq = (jax.lax.slice(x3, (0, 0), (_ROWS, _DIM))
          + jax.lax.slice(x3, (0, _DIM), (_ROWS, 2 * _DIM))) \
         + jax.lax.slice(x3, (0, 2 * _DIM), (_ROWS, 3 * _DIM))
    t = xq - z
    xq_ref[...] = z + t
    loss_ref[...] = loss_ref[...] + jnp.sum(t * t)
    ones = jnp.ones((1, _ROWS), dtype=jnp.bfloat16)
    cnt_ref[...] += jax.lax.dot_general(ones, oh, _DN,
                                        preferred_element_type=jnp.float32)

    @pl.when(step == nsteps - 1)
    def _fin():
        msq = loss_ref[...] / np.float32(nsteps * _ROWS * _DIM)
        loss_ref[...] = msq + np.float32(_BETA) * msq
        unused_ref[...] = jnp.sum((cnt_ref[...] == 0.0).astype(jnp.int32),
                                  axis=(0, 1), keepdims=True)


def kernel(x, embed):
    latent = x.reshape(-1, _DIM)
    n = latent.shape[0]
    z2 = jnp.sum(latent ** 2, axis=1, keepdims=True)
    e2 = jnp.sum(embed ** 2, axis=1)[None, :]
    ebt = embed.astype(jnp.bfloat16).T
    e_hi = embed.astype(jnp.bfloat16)
    r1 = embed - e_hi.astype(jnp.float32)
    e_mid = r1.astype(jnp.bfloat16)
    e_lo = (r1 - e_mid.astype(jnp.float32)).astype(jnp.bfloat16)
    e_cat = jnp.concatenate([e_hi, e_mid, e_lo], axis=1)
    xq, ind, loss, unused = pl.pallas_call(
        _vq_body,
        grid=(n // _ROWS,),
        in_specs=[
            pl.BlockSpec((_ROWS, _DIM), lambda i: (i, 0)),
            pl.BlockSpec((_DIM, _N_CODES), lambda i: (0, 0)),
            pl.BlockSpec((_N_CODES, 3 * _DIM), lambda i: (0, 0)),
            pl.BlockSpec((_ROWS, 1), lambda i: (i, 0)),
            pl.BlockSpec((1, _N_CODES), lambda i: (0, 0)),
        ],
        out_specs=[
            pl.BlockSpec((_ROWS, _DIM), lambda i: (i, 0)),
            pl.BlockSpec((_ROWS, 1), lambda i: (i, 0)),
            pl.BlockSpec((1, 1), lambda i: (0, 0)),
            pl.BlockSpec((1, 1), lambda i: (0, 0)),
        ],
        out_shape=[
            jax.ShapeDtypeStruct((n, _DIM), jnp.float32),
            jax.ShapeDtypeStruct((n, 1), jnp.int32),
            jax.ShapeDtypeStruct((1, 1), jnp.float32),
            jax.ShapeDtypeStruct((1, 1), jnp.int32),
        ],
        scratch_shapes=[pltpu.VMEM((1, _N_CODES), jnp.float32)],
    )(latent, ebt, e_cat, z2, e2)
    x_q_st = xq.reshape(x.shape)
    quant_loss = loss[0, 0]
    unused_codes = unused[0, 0]
    embed_ind = ind.reshape(x.shape[:-1])
    return (x_q_st, quant_loss, unused_codes, embed_ind)


# speedup vs baseline: 1.2553x; 1.0943x over previous
import jax
import jax.numpy as jnp
import numpy as np
from jax.experimental import pallas as pl
from jax.experimental.pallas import tpu as pltpu

_N_CODES = 8192
_DIM = 32
_BETA = 0.25
_ROWS = 256
_CHUNK = 2048

_DN = (((1,), (0,)), ((), ()))


def _vq_body(x_ref, ebt_ref, ecat_ref, z2_ref, e2_ref, io_ref,
             xq_ref, ind_ref, loss_ref, unused_ref, cnt_ref):
    step = pl.program_id(0)
    nsteps = pl.num_programs(0)

    @pl.when(step == 0)
    def _init():
        cnt_ref[...] = jnp.zeros_like(cnt_ref)
        loss_ref[...] = jnp.zeros_like(loss_ref)
        unused_ref[...] = jnp.zeros_like(unused_ref)

    z = x_ref[...]
    zb = (2.0 * z).astype(jnp.bfloat16)
    m = jax.lax.dot_general(zb, ebt_ref[...], _DN,
                            preferred_element_type=jnp.float32)
    dist = (z2_ref[...] - m) + e2_ref[...]

    iota_f = io_ref[...]
    acc = jnp.full((_ROWS, 1), jnp.inf, dtype=jnp.float32)
    ind = jnp.zeros((_ROWS, 1), dtype=jnp.float32)
    for k in range(_N_CODES // _CHUNK):
        dk = jax.lax.slice(dist, (0, k * _CHUNK), (_ROWS, (k + 1) * _CHUNK))
        mnk = jnp.min(dk, axis=1, keepdims=True)
        io_k = jax.lax.slice(iota_f, (0, k * _CHUNK), (1, (k + 1) * _CHUNK))
        idxk = jnp.min(jnp.where(dk == mnk, io_k, np.float32(_N_CODES)),
                       axis=1, keepdims=True)
        take = mnk < acc
        acc = jnp.where(take, mnk.astype(jnp.bfloat16).astype(jnp.float32),
                        acc)
        ind = jnp.where(take, idxk, ind)
    ind_ref[...] = ind.astype(jnp.int32)

    oh = (iota_f == ind).astype(jnp.bfloat16)
    x3 = jax.lax.dot_general(oh, ecat_ref[...], _DN,
                             preferred_element_type=jnp.float32)
    xq = (jax.lax.slice(x3, (0, 0), (_ROWS, _DIM))
          + jax.lax.slice(x3, (0, _DIM), (_ROWS, 2 * _DIM))) \
         + jax.lax.slice(x3, (0, 2 * _DIM), (_ROWS, 3 * _DIM))
    t = xq - z
    xq_ref[...] = z + t
    loss_ref[...] = loss_ref[...] + jnp.sum(t * t)
    ones = jnp.ones((1, _ROWS), dtype=jnp.bfloat16)
    cnt_ref[...] += jax.lax.dot_general(ones, oh, _DN,
                                        preferred_element_type=jnp.float32)

    @pl.when(step == nsteps - 1)
    def _fin():
        msq = loss_ref[...] / np.float32(nsteps * _ROWS * _DIM)
        loss_ref[...] = msq + np.float32(_BETA) * msq
        unused_ref[...] = jnp.sum((cnt_ref[...] == 0.0).astype(jnp.int32),
                                  axis=(0, 1), keepdims=True)


def kernel(x, embed):
    latent = x.reshape(-1, _DIM)
    n = latent.shape[0]
    z2 = jnp.sum(latent ** 2, axis=1, keepdims=True)
    e2 = jnp.sum(embed ** 2, axis=1)[None, :]
    ebt = embed.astype(jnp.bfloat16).T
    e_hi = embed.astype(jnp.bfloat16)
    r1 = embed - e_hi.astype(jnp.float32)
    e_mid = r1.astype(jnp.bfloat16)
    e_lo = (r1 - e_mid.astype(jnp.float32)).astype(jnp.bfloat16)
    e_cat = jnp.concatenate([e_hi, e_mid, e_lo], axis=1)
    xq, ind, loss, unused = pl.pallas_call(
        _vq_body,
        grid=(n // _ROWS,),
        in_specs=[
            pl.BlockSpec((_ROWS, _DIM), lambda i: (i, 0)),
            pl.BlockSpec((_DIM, _N_CODES), lambda i: (0, 0)),
            pl.BlockSpec((_N_CODES, 3 * _DIM), lambda i: (0, 0)),
            pl.BlockSpec((_ROWS, 1), lambda i: (i, 0)),
            pl.BlockSpec((1, _N_CODES), lambda i: (0, 0)),
            pl.BlockSpec((1, _N_CODES), lambda i: (0, 0)),
        ],
        out_specs=[
            pl.BlockSpec((_ROWS, _DIM), lambda i: (i, 0)),
            pl.BlockSpec((_ROWS, 1), lambda i: (i, 0)),
            pl.BlockSpec((1, 1), lambda i: (0, 0)),
            pl.BlockSpec((1, 1), lambda i: (0, 0)),
        ],
        out_shape=[
            jax.ShapeDtypeStruct((n, _DIM), jnp.float32),
            jax.ShapeDtypeStruct((n, 1), jnp.int32),
            jax.ShapeDtypeStruct((1, 1), jnp.float32),
            jax.ShapeDtypeStruct((1, 1), jnp.int32),
        ],
        scratch_shapes=[pltpu.VMEM((1, _N_CODES), jnp.float32)],
    )(latent, ebt, e_cat, z2, e2,
      jnp.arange(_N_CODES, dtype=jnp.float32)[None, :])
    x_q_st = xq.reshape(x.shape)
    quant_loss = loss[0, 0]
    unused_codes = unused[0, 0]
    embed_ind = ind.reshape(x.shape[:-1])
    return (x_q_st, quant_loss, unused_codes, embed_ind)
